# baseline (speedup 1.0000x reference)
import functools

import jax
import jax.numpy as jnp
from jax import lax
from jax.experimental import pallas as pl
from jax.experimental.pallas import tpu as pltpu
from jax.experimental.pallas import tpu_sc as plsc

EMBED = 64
BATCH = 16384
L = 16
NC, NS = 2, 16
NW = NC * NS
BPW = BATCH // NW
CHUNK = 128
NCHUNK = BPW // CHUNK
GPC = CHUNK // L
PAIR = 2 * EMBED


def _make_sc_kernel():
  mesh = plsc.VectorSubcoreMesh(core_axis_name="c", subcore_axis_name="s")

  @functools.partial(
      pl.kernel,
      mesh=mesh,
      out_type=jax.ShapeDtypeStruct((BATCH,), jnp.float32),
      compiler_params=pltpu.CompilerParams(needs_layout_passes=False),
      scratch_types=[
          pltpu.VMEM((NCHUNK, CHUNK), jnp.int32),
          pltpu.VMEM((NCHUNK, CHUNK), jnp.int32),
          pltpu.VMEM((NCHUNK, CHUNK), jnp.int32),
          pltpu.VMEM((NCHUNK, CHUNK), jnp.int32),
          pltpu.VMEM((2, CHUNK, PAIR), jnp.float32),
          pltpu.VMEM((2, CHUNK, PAIR), jnp.float32),
          pltpu.VMEM((BPW,), jnp.float32),
          pltpu.SemaphoreType.DMA,
      ],
  )
  def sc_kernel(uidx_hbm, iidx_hbm, utab_hbm, itab_hbm, out_hbm,
                uidx_v, iidx_v, urow_v, irow_v, u2_v, i2_v, out_v, sem):
    wid = lax.axis_index("s") * NC + lax.axis_index("c")
    base = wid * BPW

    pltpu.sync_copy(uidx_hbm.at[pl.ds(wid * NCHUNK, NCHUNK)], uidx_v)
    pltpu.sync_copy(iidx_hbm.at[pl.ds(wid * NCHUNK, NCHUNK)], iidx_v)
    for j in range(NCHUNK):
      for k in range(CHUNK // L):
        sl = pl.ds(k * L, L)
        urow_v[j, sl] = uidx_v[j, sl] >> 1
        irow_v[j, sl] = iidx_v[j, sl] >> 1

    def fire(j):
      b = j % 2
      return (pltpu.async_copy(utab_hbm.at[urow_v.at[j]], u2_v.at[b], sem),
              pltpu.async_copy(itab_hbm.at[irow_v.at[j]], i2_v.at[b], sem))

    lane = lax.iota(jnp.int32, L)
    inflight = fire(0)

    for j in range(NCHUNK):
      b = j % 2
      for cp in inflight:
        cp.wait()
      if j + 1 < NCHUNK:
        inflight = fire(j + 1)

      uref, iref = u2_v.at[b], i2_v.at[b]

      def group_body(g, carry, j=j, uref=uref, iref=iref):
        rows = g * L + lane
        colu = (uidx_v[j, pl.ds(g * L, L)] & 1) * EMBED
        coli = (iidx_v[j, pl.ds(g * L, L)] & 1) * EMBED
        acc = (plsc.load_gather(uref, [rows, colu]) *
               plsc.load_gather(iref, [rows, coli]))
        for c in range(1, EMBED):
          acc = acc + (plsc.load_gather(uref, [rows, colu + c]) *
                       plsc.load_gather(iref, [rows, coli + c]))
        out_v[pl.ds(j * CHUNK + g * L, L)] = acc
        return carry

      lax.fori_loop(0, GPC, group_body, 0)

    pltpu.sync_copy(out_v, out_hbm.at[pl.ds(base, BPW)])

  return sc_kernel


_SC_KERNEL = _make_sc_kernel()


def kernel(user_indices, item_indices, user_table, item_table):
  uidx = user_indices.astype(jnp.int32).reshape(NW * NCHUNK, CHUNK)
  iidx = item_indices.astype(jnp.int32).reshape(NW * NCHUNK, CHUNK)
  utab = user_table.reshape(-1, PAIR)
  itab = item_table.reshape(-1, PAIR)
  return _SC_KERNEL(uidx, iidx, utab, itab)


# speedup vs baseline: 2.7379x; 2.7379x over previous
import functools

import jax
import jax.numpy as jnp
from jax import lax
from jax.experimental import pallas as pl
from jax.experimental.pallas import tpu as pltpu
from jax.experimental.pallas import tpu_sc as plsc

EMBED = 64
BATCH = 16384
NROWS = 1000000
L = 16
NC, NS = 2, 16
FPC = EMBED // NC
BPT = BATCH // NS
CHUNK = 128
NCHUNK = BPT // CHUNK


def _make_sc_kernel():
  mesh = plsc.VectorSubcoreMesh(core_axis_name="c", subcore_axis_name="s")

  @functools.partial(
      pl.kernel,
      mesh=mesh,
      out_type=jax.ShapeDtypeStruct((NC, BATCH), jnp.float32),
      compiler_params=pltpu.CompilerParams(
          needs_layout_passes=False, use_tc_tiling_on_sc=True),
      scratch_types=[
          pltpu.VMEM((NCHUNK, CHUNK), jnp.int32),
          pltpu.VMEM((NCHUNK, CHUNK), jnp.int32),
          pltpu.VMEM_SHARED((NROWS,), jnp.float32),
          pltpu.VMEM_SHARED((NROWS,), jnp.float32),
          pltpu.VMEM((BPT,), jnp.float32),
          pltpu.VMEM((BPT,), jnp.float32),
          pltpu.VMEM((BPT,), jnp.float32),
          pltpu.SemaphoreType.DMA,
          pltpu.SemaphoreType.DMA,
      ],
  )
  def sc_kernel(uidx_hbm, iidx_hbm, utp_hbm, itp_hbm, out_hbm,
                uidx_v, iidx_v, su_sh, si_sh, uval_v, ival_v, acc_v,
                sem_s, sem_g):
    cid = lax.axis_index("c")
    sid = lax.axis_index("s")

    pltpu.sync_copy(uidx_hbm.at[pl.ds(sid * NCHUNK, NCHUNK)], uidx_v)
    pltpu.sync_copy(iidx_hbm.at[pl.ds(sid * NCHUNK, NCHUNK)], iidx_v)

    def zero_body(s, carry):
      acc_v[pl.ds(s * L, L)] = jnp.zeros((L,), jnp.float32)
      return carry

    lax.fori_loop(0, BPT // L, zero_body, 0)

    def feature_body(k, carry):
      c = cid * FPC + k

      @pl.when(sid == 0)
      def _stream():
        cu = pltpu.async_copy(utp_hbm.at[c], su_sh, sem_s)
        ci = pltpu.async_copy(itp_hbm.at[c], si_sh, sem_s)
        cu.wait()
        ci.wait()

      plsc.subcore_barrier()

      copies = []
      for q in range(NCHUNK):
        copies.append(pltpu.async_copy(
            su_sh.at[uidx_v.at[q]], uval_v.at[pl.ds(q * CHUNK, CHUNK)],
            sem_g))
        copies.append(pltpu.async_copy(
            si_sh.at[iidx_v.at[q]], ival_v.at[pl.ds(q * CHUNK, CHUNK)],
            sem_g))
      for cp in copies:
        cp.wait()

      def mac_body(s, carry):
        sl = pl.ds(s * L, L)
        acc_v[sl] = acc_v[sl] + uval_v[sl] * ival_v[sl]
        return carry

      lax.fori_loop(0, BPT // L, mac_body, 0)
      plsc.subcore_barrier()
      return carry

    lax.fori_loop(0, FPC, feature_body, 0)
    pltpu.sync_copy(acc_v, out_hbm.at[cid, pl.ds(sid * BPT, BPT)])

  return sc_kernel


_SC_KERNEL = _make_sc_kernel()


def _add_halves(x_ref, o_ref):
  o_ref[...] = x_ref[0, :] + x_ref[1, :]


def _combine(partials):
  return pl.pallas_call(
      _add_halves,
      out_shape=jax.ShapeDtypeStruct((BATCH,), jnp.float32),
  )(partials)


def kernel(user_indices, item_indices, user_table, item_table):
  uidx = user_indices.astype(jnp.int32).reshape(BATCH // CHUNK, CHUNK)
  iidx = item_indices.astype(jnp.int32).reshape(BATCH // CHUNK, CHUNK)
  partials = _SC_KERNEL(uidx, iidx, user_table.T, item_table.T)
  return _combine(partials)


# speedup vs baseline: 2.7805x; 1.0155x over previous
import functools

import jax
import jax.numpy as jnp
from jax import lax
from jax.experimental import pallas as pl
from jax.experimental.pallas import tpu as pltpu
from jax.experimental.pallas import tpu_sc as plsc

EMBED = 64
BATCH = 16384
NROWS = 1000000
L = 16
NC, NS = 2, 16
FPC = EMBED // NC
BPT = BATCH // NS
CHUNK = 128
NCHUNK = BPT // CHUNK


def _make_sc_kernel():
  mesh = plsc.VectorSubcoreMesh(core_axis_name="c", subcore_axis_name="s")

  @functools.partial(
      pl.kernel,
      mesh=mesh,
      out_type=jax.ShapeDtypeStruct((NC, BATCH), jnp.float32),
      compiler_params=pltpu.CompilerParams(
          needs_layout_passes=False, use_tc_tiling_on_sc=True),
      scratch_types=[
          pltpu.VMEM((NCHUNK, CHUNK), jnp.int32),
          pltpu.VMEM((NCHUNK, CHUNK), jnp.int32),
          pltpu.VMEM_SHARED((NROWS,), jnp.float32),
          pltpu.VMEM((BPT,), jnp.float32),
          pltpu.VMEM((BPT,), jnp.float32),
          pltpu.VMEM((BPT,), jnp.float32),
          pltpu.VMEM((EMBED, 128), jnp.float32),
          pltpu.VMEM((EMBED, 128), jnp.float32),
          pltpu.VMEM((EMBED,), jnp.int32),
          pltpu.SemaphoreType.DMA,
          pltpu.SemaphoreType.DMA,
      ],
  )
  def sc_kernel(uidx_hbm, iidx_hbm, utp_hbm, itp_hbm, utail_hbm, itail_hbm,
                out_hbm,
                uidx_v, iidx_v, s_sh, uval_v, ival_v, acc_v,
                utail_v, itail_v, tidx_v, sem_s, sem_g):
    cid = lax.axis_index("c")
    sid = lax.axis_index("s")

    pltpu.sync_copy(uidx_hbm.at[pl.ds(sid * NCHUNK, NCHUNK)], uidx_v)
    pltpu.sync_copy(iidx_hbm.at[pl.ds(sid * NCHUNK, NCHUNK)], iidx_v)

    @pl.when(sid == 15)
    def _stage_tails():
      for t in range(EMBED // L):
        tidx_v[pl.ds(t * L, L)] = lax.iota(jnp.int32, L) + t * L
      ct0 = pltpu.async_copy(utail_hbm.at[tidx_v], utail_v, sem_g)
      ct1 = pltpu.async_copy(itail_hbm.at[tidx_v], itail_v, sem_g)
      ct0.wait()
      ct1.wait()

    def zero_body(s, carry):
      acc_v[pl.ds(s * L, L)] = jnp.zeros((L,), jnp.float32)
      return carry

    lax.fori_loop(0, BPT // L, zero_body, 0)

    SLICE = 62464
    TAIL = 62976
    ALIGNED = 15 * SLICE + TAIL

    def load_row(tab_hbm, tail_v, c):
      @pl.when(sid < 15)
      def _stream_body():
        off = sid * SLICE
        cp = pltpu.async_copy(
            tab_hbm.at[c, pl.ds(off, SLICE)],
            s_sh.at[pl.ds(off, SLICE)], sem_s)
        cp.wait()

      @pl.when(sid == 15)
      def _stream_tail():
        cp = pltpu.async_copy(
            tab_hbm.at[c, pl.ds(15 * SLICE, TAIL)],
            s_sh.at[pl.ds(15 * SLICE, TAIL)], sem_s)
        ct = pltpu.async_copy(
            tail_v.at[c], s_sh.at[pl.ds(NROWS - 128, 128)], sem_s)
        cp.wait()
        ct.wait()

      plsc.subcore_barrier()

    def gather_vals(idx_v, val_v):
      copies = []
      for q in range(NCHUNK):
        copies.append(pltpu.async_copy(
            s_sh.at[idx_v.at[q]], val_v.at[pl.ds(q * CHUNK, CHUNK)],
            sem_g))
      for cp in copies:
        cp.wait()
      plsc.subcore_barrier()

    def feature_body(k, carry):
      c = cid * FPC + k
      load_row(utp_hbm, utail_v, c)
      gather_vals(uidx_v, uval_v)
      load_row(itp_hbm, itail_v, c)
      gather_vals(iidx_v, ival_v)

      def mac_body(s, carry):
        sl = pl.ds(s * L, L)
        acc_v[sl] = acc_v[sl] + uval_v[sl] * ival_v[sl]
        return carry

      lax.fori_loop(0, BPT // L, mac_body, 0)
      return carry

    lax.fori_loop(0, FPC, feature_body, 0)
    pltpu.sync_copy(acc_v, out_hbm.at[cid, pl.ds(sid * BPT, BPT)])

  return sc_kernel


_SC_KERNEL = _make_sc_kernel()


def _add_halves(x_ref, o_ref):
  o_ref[...] = x_ref[0, :] + x_ref[1, :]


def _combine(partials):
  return pl.pallas_call(
      _add_halves,
      out_shape=jax.ShapeDtypeStruct((BATCH,), jnp.float32),
  )(partials)


def kernel(user_indices, item_indices, user_table, item_table):
  uidx = user_indices.astype(jnp.int32).reshape(BATCH // CHUNK, CHUNK)
  iidx = item_indices.astype(jnp.int32).reshape(BATCH // CHUNK, CHUNK)
  utp = user_table.T
  itp = item_table.T
  partials = _SC_KERNEL(uidx, iidx, utp, itp,
                        utp[:, NROWS - 128:], itp[:, NROWS - 128:])
  return _combine(partials)


# speedup vs baseline: 2.9481x; 1.0603x over previous
import functools

import jax
import jax.numpy as jnp
from jax import lax
from jax.experimental import pallas as pl
from jax.experimental.pallas import tpu as pltpu
from jax.experimental.pallas import tpu_sc as plsc

EMBED = 64
BATCH = 16384
NROWS = 1000000
L = 16
NC, NS = 2, 16
FPC = EMBED // NC
BPT = BATCH // NS
CHUNK = 128
NCHUNK = BPT // CHUNK


def _make_sc_kernel():
  mesh = plsc.VectorSubcoreMesh(core_axis_name="c", subcore_axis_name="s")

  @functools.partial(
      pl.kernel,
      mesh=mesh,
      out_type=jax.ShapeDtypeStruct((NC, BATCH), jnp.float32),
      compiler_params=pltpu.CompilerParams(
          needs_layout_passes=False, use_tc_tiling_on_sc=True),
      scratch_types=[
          pltpu.VMEM((NCHUNK, CHUNK), jnp.int32),
          pltpu.VMEM((NCHUNK, CHUNK), jnp.int32),
          pltpu.VMEM_SHARED((NROWS,), jnp.float32),
          pltpu.VMEM_SHARED((NROWS,), jnp.float32),
          pltpu.VMEM((BPT,), jnp.float32),
          pltpu.VMEM((BPT,), jnp.float32),
          pltpu.VMEM((BPT,), jnp.float32),
          pltpu.SemaphoreType.DMA,
          pltpu.SemaphoreType.DMA,
          pltpu.SemaphoreType.DMA,
      ],
  )
  def sc_kernel(uidx_hbm, iidx_hbm, utp_hbm, itp_hbm, out_hbm,
                uidx_v, iidx_v, sa_sh, sb_sh, uval_v, ival_v, acc_v,
                sem_su, sem_si, sem_g):
    cid = lax.axis_index("c")
    sid = lax.axis_index("s")

    pltpu.sync_copy(uidx_hbm.at[pl.ds(sid * NCHUNK, NCHUNK)], uidx_v)
    pltpu.sync_copy(iidx_hbm.at[pl.ds(sid * NCHUNK, NCHUNK)], iidx_v)

    def zero_body(s, carry):
      acc_v[pl.ds(s * L, L)] = jnp.zeros((L,), jnp.float32)
      return carry

    lax.fori_loop(0, BPT // L, zero_body, 0)

    def fire_row(tab_hbm, buf_sh, c, sem):
      @pl.when(sid == 0)
      def _fire():
        pltpu.async_copy(tab_hbm.at[c], buf_sh, sem)

    def drain_row(tab_hbm, buf_sh, c, sem):
      @pl.when(sid == 0)
      def _drain():
        pltpu.make_async_copy(tab_hbm.at[c], buf_sh, sem).wait()

    def gather_vals(buf_sh, idx_v, val_v):
      copies = []
      for q in range(NCHUNK):
        copies.append(pltpu.async_copy(
            buf_sh.at[idx_v.at[q]], val_v.at[pl.ds(q * CHUNK, CHUNK)],
            sem_g))
      for cp in copies:
        cp.wait()

    fire_row(utp_hbm, sa_sh, cid * FPC, sem_su)

    def feature_body(k, carry):
      c = cid * FPC + k
      drain_row(utp_hbm, sa_sh, c, sem_su)
      plsc.subcore_barrier()
      fire_row(itp_hbm, sb_sh, c, sem_si)
      gather_vals(sa_sh, uidx_v, uval_v)
      plsc.subcore_barrier()

      @pl.when(k + 1 < FPC)
      def _prefetch():
        fire_row(utp_hbm, sa_sh, c + 1, sem_su)

      drain_row(itp_hbm, sb_sh, c, sem_si)
      plsc.subcore_barrier()
      gather_vals(sb_sh, iidx_v, ival_v)

      def mac_body(s, carry):
        sl = pl.ds(s * L, L)
        acc_v[sl] = acc_v[sl] + uval_v[sl] * ival_v[sl]
        return carry

      lax.fori_loop(0, BPT // L, mac_body, 0)
      plsc.subcore_barrier()
      return carry

    lax.fori_loop(0, FPC, feature_body, 0)
    pltpu.sync_copy(acc_v, out_hbm.at[cid, pl.ds(sid * BPT, BPT)])

  return sc_kernel


_SC_KERNEL = _make_sc_kernel()


def _add_halves(x_ref, o_ref):
  o_ref[...] = x_ref[0, :] + x_ref[1, :]


def _combine(partials):
  return pl.pallas_call(
      _add_halves,
      out_shape=jax.ShapeDtypeStruct((BATCH,), jnp.float32),
  )(partials)


def kernel(user_indices, item_indices, user_table, item_table):
  uidx = user_indices.astype(jnp.int32).reshape(BATCH // CHUNK, CHUNK)
  iidx = item_indices.astype(jnp.int32).reshape(BATCH // CHUNK, CHUNK)
  partials = _SC_KERNEL(uidx, iidx, user_table.T, item_table.T)
  return _combine(partials)


# speedup vs baseline: 3.6131x; 1.2256x over previous
import functools

import jax
import jax.numpy as jnp
from jax import lax
from jax.experimental import pallas as pl
from jax.experimental.pallas import tpu as pltpu
from jax.experimental.pallas import tpu_sc as plsc

EMBED = 64
BATCH = 16384
NROWS = 1000000
L = 16
NC, NS = 2, 16
FPC = EMBED // NC
BPT = BATCH // NS
CHUNK = 128
NCHUNK = BPT // CHUNK


def _make_sc_kernel():
  mesh = plsc.VectorSubcoreMesh(core_axis_name="c", subcore_axis_name="s")

  @functools.partial(
      pl.kernel,
      mesh=mesh,
      out_type=jax.ShapeDtypeStruct((NC, BATCH), jnp.float32),
      compiler_params=pltpu.CompilerParams(
          needs_layout_passes=False, use_tc_tiling_on_sc=True),
      scratch_types=[
          pltpu.VMEM((NCHUNK, CHUNK), jnp.int32),
          pltpu.VMEM((NCHUNK, CHUNK), jnp.int32),
          pltpu.VMEM_SHARED((NROWS,), jnp.float32),
          pltpu.VMEM_SHARED((NROWS,), jnp.float32),
          pltpu.VMEM((BPT,), jnp.float32),
          pltpu.VMEM((BPT,), jnp.float32),
          pltpu.VMEM((BPT,), jnp.float32),
          pltpu.SemaphoreType.DMA,
          pltpu.SemaphoreType.DMA,
          pltpu.SemaphoreType.DMA,
      ],
  )
  def sc_kernel(uidx_hbm, iidx_hbm, utp_hbm, itp_hbm, out_hbm,
                uidx_v, iidx_v, sa_sh, sb_sh, uval_v, ival_v, acc_v,
                sem_su, sem_si, sem_g):
    cid = lax.axis_index("c")
    sid = lax.axis_index("s")

    pltpu.sync_copy(uidx_hbm.at[pl.ds(sid * NCHUNK, NCHUNK)], uidx_v)
    pltpu.sync_copy(iidx_hbm.at[pl.ds(sid * NCHUNK, NCHUNK)], iidx_v)

    def zero_body(s, carry):
      acc_v[pl.ds(s * L, L)] = jnp.zeros((L,), jnp.float32)
      return carry

    lax.fori_loop(0, BPT // L, zero_body, 0)

    def fire_row(tab_hbm, buf_sh, c, sem):
      @pl.when(sid == 0)
      def _fire():
        pltpu.async_copy(tab_hbm.at[c], buf_sh, sem)

    def drain_row(tab_hbm, buf_sh, c, sem):
      @pl.when(sid == 0)
      def _drain():
        pltpu.make_async_copy(tab_hbm.at[c], buf_sh, sem).wait()

    def gather_vals(buf_sh, idx_v, val_v):
      copies = []
      for q in range(NCHUNK):
        copies.append(pltpu.async_copy(
            buf_sh.at[idx_v.at[q]], val_v.at[pl.ds(q * CHUNK, CHUNK)],
            sem_g))
      for cp in copies:
        cp.wait()

    fire_row(utp_hbm, sa_sh, cid * FPC, sem_su)
    fire_row(itp_hbm, sb_sh, cid * FPC, sem_si)

    def feature_body(k, carry):
      c = cid * FPC + k
      drain_row(utp_hbm, sa_sh, c, sem_su)
      plsc.subcore_barrier()
      gather_vals(sa_sh, uidx_v, uval_v)
      plsc.subcore_barrier()

      @pl.when(k + 1 < FPC)
      def _prefetch_u():
        fire_row(utp_hbm, sa_sh, c + 1, sem_su)

      drain_row(itp_hbm, sb_sh, c, sem_si)
      plsc.subcore_barrier()
      gather_vals(sb_sh, iidx_v, ival_v)

      def mac_body(s, carry):
        sl = pl.ds(s * L, L)
        acc_v[sl] = acc_v[sl] + uval_v[sl] * ival_v[sl]
        return carry

      lax.fori_loop(0, BPT // L, mac_body, 0)
      plsc.subcore_barrier()

      @pl.when(k + 1 < FPC)
      def _prefetch_i():
        fire_row(itp_hbm, sb_sh, c + 1, sem_si)

      return carry

    lax.fori_loop(0, FPC, feature_body, 0)
    pltpu.sync_copy(acc_v, out_hbm.at[cid, pl.ds(sid * BPT, BPT)])

  return sc_kernel


_SC_KERNEL = _make_sc_kernel()


def _add_halves(x_ref, o_ref):
  o_ref[...] = x_ref[0, :] + x_ref[1, :]


def _combine(partials):
  return pl.pallas_call(
      _add_halves,
      out_shape=jax.ShapeDtypeStruct((BATCH,), jnp.float32),
  )(partials)


def kernel(user_indices, item_indices, user_table, item_table):
  uidx = user_indices.astype(jnp.int32).reshape(BATCH // CHUNK, CHUNK)
  iidx = item_indices.astype(jnp.int32).reshape(BATCH // CHUNK, CHUNK)
  partials = _SC_KERNEL(uidx, iidx, user_table.T, item_table.T)
  return _combine(partials)


# speedup vs baseline: 3.6153x; 1.0006x over previous
import functools

import jax
import jax.numpy as jnp
from jax import lax
from jax.experimental import pallas as pl
from jax.experimental.pallas import tpu as pltpu
from jax.experimental.pallas import tpu_sc as plsc

EMBED = 64
BATCH = 16384
NROWS = 1000000
L = 16
NC, NS = 2, 16
FPC = EMBED // NC
BPT = BATCH // NS
CHUNK = 128
NCHUNK = BPT // CHUNK


def _make_sc_kernel():
  mesh = plsc.VectorSubcoreMesh(core_axis_name="c", subcore_axis_name="s")

  @functools.partial(
      pl.kernel,
      mesh=mesh,
      out_type=jax.ShapeDtypeStruct((NC, BATCH), jnp.float32),
      compiler_params=pltpu.CompilerParams(
          needs_layout_passes=False, use_tc_tiling_on_sc=True),
      scratch_types=[
          pltpu.VMEM((NCHUNK, CHUNK), jnp.int32),
          pltpu.VMEM((NCHUNK, CHUNK), jnp.int32),
          pltpu.VMEM_SHARED((NROWS,), jnp.float32),
          pltpu.VMEM_SHARED((NROWS,), jnp.float32),
          pltpu.VMEM((BPT,), jnp.float32),
          pltpu.VMEM((BPT,), jnp.float32),
          pltpu.VMEM((BPT,), jnp.float32),
          pltpu.SemaphoreType.DMA,
          pltpu.SemaphoreType.DMA,
          pltpu.SemaphoreType.DMA,
      ],
  )
  def sc_kernel(uidx_hbm, iidx_hbm, utp_hbm, itp_hbm, out_hbm,
                uidx_v, iidx_v, sa_sh, sb_sh, uval_v, ival_v, acc_v,
                sem_su, sem_si, sem_g):
    cid = lax.axis_index("c")
    sid = lax.axis_index("s")

    pltpu.sync_copy(uidx_hbm.at[pl.ds(sid * NCHUNK, NCHUNK)], uidx_v)
    pltpu.sync_copy(iidx_hbm.at[pl.ds(sid * NCHUNK, NCHUNK)], iidx_v)

    def zero_body(s, carry):
      acc_v[pl.ds(s * L, L)] = jnp.zeros((L,), jnp.float32)
      return carry

    lax.fori_loop(0, BPT // L, zero_body, 0)

    def fire_row(tab_hbm, buf_sh, c, sem):
      @pl.when(sid == 0)
      def _fire():
        pltpu.async_copy(tab_hbm.at[c], buf_sh, sem)

    def drain_row(tab_hbm, buf_sh, c, sem):
      @pl.when(sid == 0)
      def _drain():
        pltpu.make_async_copy(tab_hbm.at[c], buf_sh, sem).wait()

    def gather_vals(buf_sh, idx_v, val_v):
      copies = []
      for q in range(NCHUNK):
        copies.append(pltpu.async_copy(
            buf_sh.at[idx_v.at[q]], val_v.at[pl.ds(q * CHUNK, CHUNK)],
            sem_g))
      for cp in copies:
        cp.wait()

    fire_row(utp_hbm, sa_sh, cid * FPC, sem_su)
    fire_row(itp_hbm, sb_sh, cid * FPC, sem_si)

    def feature_body(k, carry):
      c = cid * FPC + k
      drain_row(utp_hbm, sa_sh, c, sem_su)
      plsc.subcore_barrier()
      gather_vals(sa_sh, uidx_v, uval_v)
      plsc.subcore_barrier()

      @pl.when(k + 1 < FPC)
      def _prefetch_u():
        fire_row(utp_hbm, sa_sh, c + 1, sem_su)

      drain_row(itp_hbm, sb_sh, c, sem_si)
      plsc.subcore_barrier()
      gather_vals(sb_sh, iidx_v, ival_v)
      plsc.subcore_barrier()

      @pl.when(k + 1 < FPC)
      def _prefetch_i():
        fire_row(itp_hbm, sb_sh, c + 1, sem_si)

      def mac_body(s, carry):
        sl = pl.ds(s * L, L)
        acc_v[sl] = acc_v[sl] + uval_v[sl] * ival_v[sl]
        return carry

      lax.fori_loop(0, BPT // L, mac_body, 0)
      return carry

    lax.fori_loop(0, FPC, feature_body, 0)
    pltpu.sync_copy(acc_v, out_hbm.at[cid, pl.ds(sid * BPT, BPT)])

  return sc_kernel


_SC_KERNEL = _make_sc_kernel()


def _add_halves(x_ref, o_ref):
  o_ref[...] = x_ref[0, :] + x_ref[1, :]


def _combine(partials):
  return pl.pallas_call(
      _add_halves,
      out_shape=jax.ShapeDtypeStruct((BATCH,), jnp.float32),
  )(partials)


def kernel(user_indices, item_indices, user_table, item_table):
  uidx = user_indices.astype(jnp.int32).reshape(BATCH // CHUNK, CHUNK)
  iidx = item_indices.astype(jnp.int32).reshape(BATCH // CHUNK, CHUNK)
  partials = _SC_KERNEL(uidx, iidx, user_table.T, item_table.T)
  return _combine(partials)
